# baseline (speedup 1.0000x reference)
import numpy as np
import jax
import jax.numpy as jnp
from jax import lax
from jax.experimental import pallas as pl
from jax.experimental.pallas import tpu as pltpu

ROWS = 16384
CLASSES = 1000
BLOCK_ROWS = 256
NBLK = ROWS // BLOCK_ROWS
NUM_BINS = 15

_BOUNDS = np.linspace(0.0, 1.0, NUM_BINS + 1)


def _rowstats_body(x_ref, lab_ref, conf_ref, acc_ref):
    x = x_ref[...]
    m = jnp.max(x, axis=1, keepdims=True)
    s = jnp.sum(jnp.exp(x - m), axis=1)
    idx = lax.broadcasted_iota(jnp.int32, x.shape, 1)
    pred = jnp.min(jnp.where(x == m, idx, CLASSES), axis=1)
    conf_ref[0, 0, :] = 1.0 / s
    acc_ref[0, 0, :] = (pred == lab_ref[0, 0, :]).astype(jnp.float32)


def _ece_body(conf_ref, acc_ref, out_ref):
    conf = conf_ref[...]
    acc = acc_ref[...]
    total = float(ROWS)
    ece = jnp.float32(0.0)
    for b in range(NUM_BINS):
        inb = (conf > jnp.float32(_BOUNDS[b])) & (conf <= jnp.float32(_BOUNDS[b + 1]))
        inbf = inb.astype(jnp.float32)
        cnt = jnp.sum(inbf)
        sc = jnp.sum(inbf * conf)
        sa = jnp.sum(inbf * acc)
        safe = jnp.maximum(cnt, 1.0)
        ece = ece + jnp.where(
            cnt > 0.0, jnp.abs(sc / safe - sa / safe) * (cnt / total), 0.0
        )
    out_ref[...] = jnp.full((1, 1), ece, jnp.float32)


def _rowstats(inputs, labs):
    return pl.pallas_call(
        _rowstats_body,
        grid=(NBLK,),
        in_specs=[
            pl.BlockSpec((BLOCK_ROWS, CLASSES), lambda i: (i, 0)),
            pl.BlockSpec((1, 1, BLOCK_ROWS), lambda i: (i, 0, 0)),
        ],
        out_specs=[
            pl.BlockSpec((1, 1, BLOCK_ROWS), lambda i: (i, 0, 0)),
            pl.BlockSpec((1, 1, BLOCK_ROWS), lambda i: (i, 0, 0)),
        ],
        out_shape=[
            jax.ShapeDtypeStruct((NBLK, 1, BLOCK_ROWS), jnp.float32),
            jax.ShapeDtypeStruct((NBLK, 1, BLOCK_ROWS), jnp.float32),
        ],
    )(inputs, labs)


def _ece_reduce(conf2, acc2):
    return pl.pallas_call(
        _ece_body,
        out_shape=jax.ShapeDtypeStruct((1, 1), jnp.float32),
    )(conf2, acc2)


def kernel(inputs, labels):
    labs = labels.reshape(NBLK, 1, BLOCK_ROWS)
    conf, acc = _rowstats(inputs, labs)
    return (jnp.sum(conf) + jnp.sum(acc)).reshape(1)


# speedup vs baseline: 3.7909x; 3.7909x over previous
import numpy as np
import jax
import jax.numpy as jnp
from jax import lax
from jax.experimental import pallas as pl
from jax.experimental.pallas import tpu as pltpu

ROWS = 16384
CLASSES = 1000
BLOCK_COLS = 1024
NBLK = ROWS // BLOCK_COLS
NUM_BINS = 15

_BOUNDS = np.linspace(0.0, 1.0, NUM_BINS + 1)


def _rowstats_body(xt_ref, lab_ref, conf_ref, acc_ref):
    x = xt_ref[...]
    m = jnp.max(x, axis=0, keepdims=True)
    s = jnp.sum(jnp.exp(x - m), axis=0)
    idx = lax.broadcasted_iota(jnp.int32, x.shape, 0)
    pred = jnp.min(jnp.where(x == m, idx, CLASSES), axis=0)
    conf_ref[0, 0, :] = 1.0 / s
    acc_ref[0, 0, :] = (pred == lab_ref[0, 0, :]).astype(jnp.float32)


def _ece_body(conf_ref, acc_ref, out_ref):
    conf = conf_ref[...]
    acc = acc_ref[...]
    total = float(ROWS)
    ece = jnp.float32(0.0)
    for b in range(NUM_BINS):
        inb = (conf > jnp.float32(_BOUNDS[b])) & (conf <= jnp.float32(_BOUNDS[b + 1]))
        inbf = inb.astype(jnp.float32)
        cnt = jnp.sum(inbf)
        sc = jnp.sum(inbf * conf)
        sa = jnp.sum(inbf * acc)
        safe = jnp.maximum(cnt, 1.0)
        ece = ece + jnp.where(
            cnt > 0.0, jnp.abs(sc / safe - sa / safe) * (cnt / total), 0.0
        )
    out_ref[...] = jnp.full((1, 1), ece, jnp.float32)


def _rowstats(xt, labs):
    return pl.pallas_call(
        _rowstats_body,
        grid=(NBLK,),
        in_specs=[
            pl.BlockSpec((CLASSES, BLOCK_COLS), lambda i: (0, i)),
            pl.BlockSpec((1, 1, BLOCK_COLS), lambda i: (i, 0, 0)),
        ],
        out_specs=[
            pl.BlockSpec((1, 1, BLOCK_COLS), lambda i: (i, 0, 0)),
            pl.BlockSpec((1, 1, BLOCK_COLS), lambda i: (i, 0, 0)),
        ],
        out_shape=[
            jax.ShapeDtypeStruct((NBLK, 1, BLOCK_COLS), jnp.float32),
            jax.ShapeDtypeStruct((NBLK, 1, BLOCK_COLS), jnp.float32),
        ],
    )(xt, labs)


def _ece_reduce(conf2, acc2):
    return pl.pallas_call(
        _ece_body,
        out_shape=jax.ShapeDtypeStruct((1, 1), jnp.float32),
    )(conf2, acc2)


def kernel(inputs, labels):
    labs = labels.reshape(NBLK, 1, BLOCK_COLS)
    conf, acc = _rowstats(inputs.T, labs)
    conf2 = conf.reshape(128, 128)
    acc2 = acc.reshape(128, 128)
    ece = _ece_reduce(conf2, acc2)
    return ece.reshape(1)
